# initial kernel scaffold (unmeasured)
import jax
import jax.numpy as jnp
from jax import lax
from jax.experimental import pallas as pl
from jax.experimental.pallas import tpu as pltpu


def kernel(
    x,
):
    def body(*refs):
        pass

    out_shape = jax.ShapeDtypeStruct(..., jnp.float32)
    return pl.pallas_call(body, out_shape=out_shape)(...)


# baseline (device time: 19671 ns/iter reference)
import jax
import jax.numpy as jnp
from jax import lax
from jax.experimental import pallas as pl
from jax.experimental.pallas import tpu as pltpu

N_DEV = 8
_STRIDES = (1, 4, 2)


def kernel(x):
    m, n = x.shape

    def body(x_ref, out_ref, comm_ref, send_sems, recv_sems):
        my = lax.axis_index("i")

        barrier_sem = pltpu.get_barrier_semaphore()
        for s in _STRIDES:
            pl.semaphore_signal(
                barrier_sem, inc=1,
                device_id=(my ^ s,), device_id_type=pl.DeviceIdType.MESH,
            )
        pl.semaphore_wait(barrier_sem, len(_STRIDES))

        out_ref[...] = x_ref[...]
        for step, s in enumerate(_STRIDES):
            rdma = pltpu.make_async_remote_copy(
                src_ref=out_ref,
                dst_ref=comm_ref.at[step],
                send_sem=send_sems.at[step],
                recv_sem=recv_sems.at[step],
                device_id=(my ^ s,),
                device_id_type=pl.DeviceIdType.MESH,
            )
            rdma.start()
            rdma.wait()
            out_ref[...] += comm_ref[step]

    return pl.pallas_call(
        body,
        out_shape=jax.ShapeDtypeStruct((m, n), x.dtype),
        in_specs=[pl.BlockSpec(memory_space=pltpu.VMEM)],
        out_specs=pl.BlockSpec(memory_space=pltpu.VMEM),
        scratch_shapes=[
            pltpu.VMEM((len(_STRIDES), m, n), x.dtype),
            pltpu.SemaphoreType.DMA((len(_STRIDES),)),
            pltpu.SemaphoreType.DMA((len(_STRIDES),)),
        ],
        compiler_params=pltpu.CompilerParams(collective_id=0),
    )(x)


# device time: 18805 ns/iter; 1.0461x vs baseline; 1.0461x over previous
import jax
import jax.numpy as jnp
from jax import lax
from jax.experimental import pallas as pl
from jax.experimental.pallas import tpu as pltpu

N_DEV = 8
_STRIDES = (1, 3, 4)


def kernel(x):
    m, n = x.shape

    def body(x_ref, out_ref, comm_ref, send_sems, recv_sems):
        my = lax.axis_index("i")

        barrier_sem = pltpu.get_barrier_semaphore()
        for s in _STRIDES:
            pl.semaphore_signal(
                barrier_sem, inc=1,
                device_id=(my ^ s,), device_id_type=pl.DeviceIdType.MESH,
            )
        pl.semaphore_wait(barrier_sem, len(_STRIDES))

        out_ref[...] = x_ref[...]
        for step, s in enumerate(_STRIDES):
            rdma = pltpu.make_async_remote_copy(
                src_ref=out_ref,
                dst_ref=comm_ref.at[step],
                send_sem=send_sems.at[step],
                recv_sem=recv_sems.at[step],
                device_id=(my ^ s,),
                device_id_type=pl.DeviceIdType.MESH,
            )
            rdma.start()
            rdma.wait()
            out_ref[...] += comm_ref[step]

    return pl.pallas_call(
        body,
        out_shape=jax.ShapeDtypeStruct((m, n), x.dtype),
        in_specs=[pl.BlockSpec(memory_space=pltpu.VMEM)],
        out_specs=pl.BlockSpec(memory_space=pltpu.VMEM),
        scratch_shapes=[
            pltpu.VMEM((len(_STRIDES), m, n), x.dtype),
            pltpu.SemaphoreType.DMA((len(_STRIDES),)),
            pltpu.SemaphoreType.DMA((len(_STRIDES),)),
        ],
        compiler_params=pltpu.CompilerParams(collective_id=0),
    )(x)


# device time: 14626 ns/iter; 1.3449x vs baseline; 1.2857x over previous
import jax
import jax.numpy as jnp
from jax import lax
from jax.experimental import pallas as pl
from jax.experimental.pallas import tpu as pltpu

N_DEV = 8
_ORDERS = ((1, 3, 4), (3, 4, 1))
_NSTEP = 3


def kernel(x):
    m, n = x.shape
    half = m // 2

    def body(x_ref, out_ref, comm_ref, send_sems, recv_sems):
        my = lax.axis_index("i")

        barrier_sem = pltpu.get_barrier_semaphore()
        for s in (1, 3, 4):
            pl.semaphore_signal(
                barrier_sem, inc=1,
                device_id=(my ^ s,), device_id_type=pl.DeviceIdType.MESH,
            )
        pl.semaphore_wait(barrier_sem, 3)

        out_ref[...] = x_ref[...]
        for t in range(_NSTEP):
            rdmas = []
            for h in range(2):
                rdma = pltpu.make_async_remote_copy(
                    src_ref=out_ref.at[pl.ds(h * half, half), :],
                    dst_ref=comm_ref.at[t, h],
                    send_sem=send_sems.at[t, h],
                    recv_sem=recv_sems.at[t, h],
                    device_id=(my ^ _ORDERS[h][t],),
                    device_id_type=pl.DeviceIdType.MESH,
                )
                rdma.start()
                rdmas.append(rdma)
            for h in range(2):
                rdmas[h].wait()
                out_ref[pl.ds(h * half, half), :] += comm_ref[t, h]

    return pl.pallas_call(
        body,
        out_shape=jax.ShapeDtypeStruct((m, n), x.dtype),
        in_specs=[pl.BlockSpec(memory_space=pltpu.VMEM)],
        out_specs=pl.BlockSpec(memory_space=pltpu.VMEM),
        scratch_shapes=[
            pltpu.VMEM((_NSTEP, 2, half, n), x.dtype),
            pltpu.SemaphoreType.DMA((_NSTEP, 2)),
            pltpu.SemaphoreType.DMA((_NSTEP, 2)),
        ],
        compiler_params=pltpu.CompilerParams(collective_id=0),
    )(x)


# device time: 13360 ns/iter; 1.4724x vs baseline; 1.0948x over previous
import jax
import jax.numpy as jnp
from jax import lax
from jax.experimental import pallas as pl
from jax.experimental.pallas import tpu as pltpu

N_DEV = 8
_ORDERS = ((1, 3, 4), (3, 4, 1), (4, 1, 3))
_ROWS = (88, 88, 80)
_OFFS = (0, 88, 176)
_NSTEP = 3
_NCHUNK = 3


def kernel(x):
    m, n = x.shape

    def body(x_ref, out_ref, comm_ref, send_sems, recv_sems):
        my = lax.axis_index("i")

        barrier_sem = pltpu.get_barrier_semaphore()
        for s in (1, 3, 4):
            pl.semaphore_signal(
                barrier_sem, inc=1,
                device_id=(my ^ s,), device_id_type=pl.DeviceIdType.MESH,
            )
        pl.semaphore_wait(barrier_sem, 3)

        out_ref[...] = x_ref[...]

        def make_rdma(t, k):
            return pltpu.make_async_remote_copy(
                src_ref=out_ref.at[pl.ds(_OFFS[k], _ROWS[k]), :],
                dst_ref=comm_ref.at[t, k, pl.ds(0, _ROWS[k]), :],
                send_sem=send_sems.at[t, k],
                recv_sem=recv_sems.at[t, k],
                device_id=(my ^ _ORDERS[k][t],),
                device_id_type=pl.DeviceIdType.MESH,
            )

        rdmas = [make_rdma(0, k) for k in range(_NCHUNK)]
        for r in rdmas:
            r.start()
        for t in range(_NSTEP):
            for k in range(_NCHUNK):
                rdmas[k].wait()
                out_ref[pl.ds(_OFFS[k], _ROWS[k]), :] += (
                    comm_ref[t, k, pl.ds(0, _ROWS[k]), :]
                )
                if t + 1 < _NSTEP:
                    rdmas[k] = make_rdma(t + 1, k)
                    rdmas[k].start()

    return pl.pallas_call(
        body,
        out_shape=jax.ShapeDtypeStruct((m, n), x.dtype),
        in_specs=[pl.BlockSpec(memory_space=pltpu.VMEM)],
        out_specs=pl.BlockSpec(memory_space=pltpu.VMEM),
        scratch_shapes=[
            pltpu.VMEM((_NSTEP, _NCHUNK, max(_ROWS), n), x.dtype),
            pltpu.SemaphoreType.DMA((_NSTEP, _NCHUNK)),
            pltpu.SemaphoreType.DMA((_NSTEP, _NCHUNK)),
        ],
        compiler_params=pltpu.CompilerParams(collective_id=0),
    )(x)


# device time: 12076 ns/iter; 1.6289x vs baseline; 1.1063x over previous
import jax
import jax.numpy as jnp
from jax import lax
from jax.experimental import pallas as pl
from jax.experimental.pallas import tpu as pltpu

N_DEV = 8


def kernel(x):
    m, n = x.shape
    piece = m // N_DEV

    def body(x_ref, out_ref, rs_ref, rs_send, rs_recv, ag_send, ag_recv):
        my = lax.axis_index("i")

        barrier_sem = pltpu.get_barrier_semaphore()
        for r in range(1, N_DEV):
            pl.semaphore_signal(
                barrier_sem, inc=1,
                device_id=(my ^ r,), device_id_type=pl.DeviceIdType.MESH,
            )
        pl.semaphore_wait(barrier_sem, N_DEV - 1)

        rs = []
        for r in range(1, N_DEV):
            tgt = my ^ r
            rdma = pltpu.make_async_remote_copy(
                src_ref=x_ref.at[pl.ds(tgt * piece, piece), :],
                dst_ref=rs_ref.at[r],
                send_sem=rs_send.at[r],
                recv_sem=rs_recv.at[r],
                device_id=(tgt,),
                device_id_type=pl.DeviceIdType.MESH,
            )
            rdma.start()
            rs.append(rdma)

        acc = x_ref[pl.ds(my * piece, piece), :]
        for r in range(1, N_DEV):
            rs[r - 1].wait()
            acc = acc + rs_ref[r]
        out_ref[pl.ds(my * piece, piece), :] = acc

        ag = []
        for r in range(1, N_DEV):
            tgt = my ^ r
            rdma = pltpu.make_async_remote_copy(
                src_ref=out_ref.at[pl.ds(my * piece, piece), :],
                dst_ref=out_ref.at[pl.ds(my * piece, piece), :],
                send_sem=ag_send.at[r],
                recv_sem=ag_recv.at[r],
                device_id=(tgt,),
                device_id_type=pl.DeviceIdType.MESH,
            )
            rdma.start()
            ag.append(rdma)
        for r in range(1, N_DEV):
            ag[r - 1].wait()

    return pl.pallas_call(
        body,
        out_shape=jax.ShapeDtypeStruct((m, n), x.dtype),
        in_specs=[pl.BlockSpec(memory_space=pltpu.VMEM)],
        out_specs=pl.BlockSpec(memory_space=pltpu.VMEM),
        scratch_shapes=[
            pltpu.VMEM((N_DEV, piece, n), x.dtype),
            pltpu.SemaphoreType.DMA((N_DEV,)),
            pltpu.SemaphoreType.DMA((N_DEV,)),
            pltpu.SemaphoreType.DMA((N_DEV,)),
            pltpu.SemaphoreType.DMA((N_DEV,)),
        ],
        compiler_params=pltpu.CompilerParams(collective_id=0),
    )(x)
